# baseline (speedup 1.0000x reference)
import functools

import jax
import jax.numpy as jnp
from jax import lax
from jax.experimental import pallas as pl
from jax.experimental.pallas import tpu as pltpu
from jax.experimental.pallas import tpu_sc as plsc

VOCAB = 100000
DIM = 200
DPAD = 256
WPAD = DPAD // 2
B = 4096
N_POS = 20
N_NEG = 200
K = 224
KH = K // 2
NBUF = 2
NC = 2
NS = 16
NW = NC * NS
BPW = B // NW
BBLK = 64
LANES = 16
NCHUNK = WPAD // LANES


_mesh = plsc.VectorSubcoreMesh(core_axis_name="c", subcore_axis_name="s")


@functools.partial(
    pl.kernel,
    mesh=_mesh,
    out_type=jax.ShapeDtypeStruct((B * K,), jnp.float32),
    compiler_params=pltpu.CompilerParams(
        needs_layout_passes=False, use_tc_tiling_on_sc=True),
    scratch_types=[
        pltpu.VMEM((BPW, WPAD), jnp.int32),
        pltpu.VMEM((BBLK, K), jnp.int32),
        [pltpu.VMEM((KH,), jnp.int32) for _ in range(NBUF)],
        pltpu.VMEM((NBUF, KH, WPAD), jnp.int32),
        pltpu.VMEM((BBLK * K,), jnp.float32),
        pltpu.SemaphoreType.DMA((NBUF,)),
    ],
)
def _sc_scores(idx_hbm, ctr_hbm, out_t_hbm, out_hbm,
               crows_v, kidx_v, gidx_list, rows_v, sc_v, gsem):
    gidx_v = list(gidx_list)
    wid = lax.axis_index("s") * NC + lax.axis_index("c")
    lane = lax.iota(jnp.int32, 16)
    lane0 = lane == 0

    pltpu.sync_copy(ctr_hbm.at[pl.ds(wid * BPW, BPW)], crows_v)

    def fire(b1, q):
        h1 = q & 1
        gb = gidx_v[q]
        for g in range(KH // 16):
            gb[pl.ds(g * 16, 16)] = kidx_v[b1, pl.ds(h1 * KH + g * 16, 16)]
        pltpu.async_copy(out_t_hbm.at[gb], rows_v.at[q], gsem.at[q])

    def drain(q):
        pltpu.make_async_copy(out_t_hbm.at[gidx_v[q]], rows_v.at[q],
                              gsem.at[q]).wait()

    def compute(c, b, q):
        bc = c * BBLK + b
        h = q & 1
        cvec = [plsc.bitcast(crows_v[bc, pl.ds(u * LANES, LANES)],
                             jnp.bfloat16)
                for u in range(NCHUNK)]
        obase = jnp.full((16,), b * K + h * KH, jnp.int32)

        def row(j, _):
            r0 = plsc.bitcast(rows_v[q, j, pl.ds(0, LANES)], jnp.bfloat16)
            acc = r0 * cvec[0]
            for u in range(1, NCHUNK):
                ru = plsc.bitcast(rows_v[q, j, pl.ds(u * LANES, LANES)],
                                  jnp.bfloat16)
                acc += ru * cvec[u]
            ev, od = plsc.unpack(acc, format=plsc.PackFormat.INTERLEAVED)
            s = jnp.sum(ev + od)
            plsc.store_scatter(sc_v, [obase + j],
                               jnp.full((16,), s), mask=lane0)
            return 0

        lax.fori_loop(0, KH, row, 0, unroll=4)

    for c in range(BPW // BBLK):
        base = wid * BPW + c * BBLK

        pltpu.sync_copy(idx_hbm.at[pl.ds(base, BBLK)], kidx_v)
        fire(0, 0)

        def body(b, _):
            fire(b, 1)
            drain(0)
            compute(c, b, 0)

            @pl.when(b < BBLK - 1)
            def _next():
                fire(b + 1, 0)

            drain(1)
            compute(c, b, 1)
            return 0

        lax.fori_loop(0, BBLK, body, 0)
        pltpu.sync_copy(sc_v, out_hbm.at[pl.ds(base * K, BBLK * K)])


NROW_TC = B * K // 128
BLK_TC = NROW_TC // 8


def _loss_body(scores_ref, out_ref):
    i = pl.program_id(0)

    @pl.when(i == 0)
    def _init():
        out_ref[...] = jnp.zeros((1, 1), jnp.float32)

    x = scores_ref[...]
    flat = (i * BLK_TC * 128
            + lax.broadcasted_iota(jnp.int32, x.shape, 0) * 128
            + lax.broadcasted_iota(jnp.int32, x.shape, 1))
    col = flat % K
    x = jnp.where(col < N_POS, x, -x)
    ls = jnp.where(col < N_POS + N_NEG, jax.nn.log_sigmoid(x), 0.0)
    out_ref[...] += jnp.sum(ls).reshape(1, 1)

    @pl.when(i == pl.num_programs(0) - 1)
    def _fini():
        out_ref[...] = -out_ref[...] / B


def _pack_rows(t):
    n = t.shape[0]
    lo = t[:, :WPAD]
    hi = jnp.concatenate(
        [t[:, WPAD:], jnp.zeros((n, DPAD - DIM), t.dtype)], axis=1)

    def rne(x):
        b = lax.bitcast_convert_type(x, jnp.uint32)
        return b + 0x7FFF + ((b >> 16) & 1)

    packed = (rne(lo) >> 16) | (rne(hi) & jnp.uint32(0xFFFF0000))
    return lax.bitcast_convert_type(packed, jnp.int32)


def kernel(center_word, pos_words, neg_words, in_table, out_table):
    idx_all = jnp.concatenate(
        [pos_words, neg_words,
         jnp.zeros((B, K - N_POS - N_NEG), jnp.int32)], axis=1)
    ctr_pk = _pack_rows(jnp.take(in_table, center_word, axis=0))
    out_pk = _pack_rows(out_table)

    scores = _sc_scores(idx_all, ctr_pk, out_pk)
    scores = scores.reshape(NROW_TC, 128)

    loss = pl.pallas_call(
        _loss_body,
        grid=(NROW_TC // BLK_TC,),
        in_specs=[pl.BlockSpec((BLK_TC, 128), lambda i: (i, 0))],
        out_specs=pl.BlockSpec((1, 1), lambda i: (0, 0)),
        out_shape=jax.ShapeDtypeStruct((1, 1), jnp.float32),
    )(scores)
    return loss[0, 0]


# speedup vs baseline: 1.3529x; 1.3529x over previous
import functools

import jax
import jax.numpy as jnp
from jax import lax
from jax.experimental import pallas as pl
from jax.experimental.pallas import tpu as pltpu
from jax.experimental.pallas import tpu_sc as plsc

VOCAB = 100000
DIM = 200
DPAD = 256
WPAD = DPAD // 2
B = 4096
N_POS = 20
N_NEG = 200
K = 224
KH = K // 2
NBUF = 2
NC = 2
NS = 16
NW = NC * NS
BPW = B // NW
BBLK = 64
LANES = 16
NCHUNK = WPAD // LANES


_mesh = plsc.VectorSubcoreMesh(core_axis_name="c", subcore_axis_name="s")


@functools.partial(
    pl.kernel,
    mesh=_mesh,
    out_type=jax.ShapeDtypeStruct((B * K,), jnp.float32),
    compiler_params=pltpu.CompilerParams(
        needs_layout_passes=False, use_tc_tiling_on_sc=True),
    scratch_types=[
        pltpu.VMEM((BPW, WPAD), jnp.int32),
        pltpu.VMEM((BBLK, K), jnp.int32),
        [pltpu.VMEM((KH,), jnp.int32) for _ in range(NBUF)],
        pltpu.VMEM((NBUF, KH, WPAD), jnp.int32),
        pltpu.VMEM((BBLK * K,), jnp.float32),
        pltpu.SemaphoreType.DMA((NBUF,)),
    ],
)
def _sc_scores(idx_hbm, ctr_hbm, out_t_hbm, out_hbm,
               crows_v, kidx_v, gidx_list, rows_v, sc_v, gsem):
    gidx_v = list(gidx_list)
    wid = lax.axis_index("s") * NC + lax.axis_index("c")
    lane = lax.iota(jnp.int32, 16)
    lane0 = lane == 0

    pltpu.sync_copy(ctr_hbm.at[pl.ds(wid * BPW, BPW)], crows_v)

    def fire(b1, q):
        h1 = q & 1
        gb = gidx_v[q]
        for g in range(KH // 16):
            gb[pl.ds(g * 16, 16)] = kidx_v[b1, pl.ds(h1 * KH + g * 16, 16)]
        pltpu.async_copy(out_t_hbm.at[gb], rows_v.at[q], gsem.at[q])

    def drain(q):
        pltpu.make_async_copy(out_t_hbm.at[gidx_v[q]], rows_v.at[q],
                              gsem.at[q]).wait()

    def compute(c, b, q):
        bc = c * BBLK + b
        h = q & 1
        cvec = [plsc.bitcast(crows_v[bc, pl.ds(u * LANES, LANES)],
                             jnp.bfloat16)
                for u in range(NCHUNK)]
        obase = jnp.full((16,), b * K + h * KH, jnp.int32)

        def row(j, _):
            r0 = plsc.bitcast(rows_v[q, j, pl.ds(0, LANES)], jnp.bfloat16)
            acc = r0 * cvec[0]
            for u in range(1, NCHUNK):
                ru = plsc.bitcast(rows_v[q, j, pl.ds(u * LANES, LANES)],
                                  jnp.bfloat16)
                acc += ru * cvec[u]
            ev, od = plsc.unpack(acc, format=plsc.PackFormat.INTERLEAVED)
            s = jnp.sum(ev + od)
            plsc.store_scatter(sc_v, [obase + j],
                               jnp.full((16,), s), mask=lane0)
            return 0

        lax.fori_loop(0, KH, row, 0, unroll=4)

    for c in range(BPW // BBLK):
        base = wid * BPW + c * BBLK

        pltpu.sync_copy(idx_hbm.at[pl.ds(base, BBLK)], kidx_v)
        fire(0, 0)

        def body(b, _):
            fire(b, 1)
            drain(0)
            compute(c, b, 0)

            @pl.when(b < BBLK - 1)
            def _next():
                fire(b + 1, 0)

            drain(1)
            compute(c, b, 1)
            return 0

        lax.fori_loop(0, BBLK, body, 0)
        pltpu.sync_copy(sc_v, out_hbm.at[pl.ds(base * K, BBLK * K)])


NROW_TC = B * K // 128
BLK_TC = NROW_TC // 8


def _loss_body(scores_ref, out_ref):
    i = pl.program_id(0)

    @pl.when(i == 0)
    def _init():
        out_ref[...] = jnp.zeros((1, 1), jnp.float32)

    x = scores_ref[...]
    flat = (i * BLK_TC * 128
            + lax.broadcasted_iota(jnp.int32, x.shape, 0) * 128
            + lax.broadcasted_iota(jnp.int32, x.shape, 1))
    col = flat % K
    x = jnp.where(col < N_POS, x, -x)
    ls = jnp.where(col < N_POS + N_NEG, jax.nn.log_sigmoid(x), 0.0)
    out_ref[...] += jnp.sum(ls).reshape(1, 1)

    @pl.when(i == pl.num_programs(0) - 1)
    def _fini():
        out_ref[...] = -out_ref[...] / B


def _pack_rows(t):
    n = t.shape[0]
    lo = t[:, :WPAD].astype(jnp.bfloat16)
    hi = jnp.concatenate(
        [t[:, WPAD:].astype(jnp.bfloat16),
         jnp.zeros((n, DPAD - DIM), jnp.bfloat16)], axis=1)
    lo16 = lax.bitcast_convert_type(lo, jnp.uint16).astype(jnp.uint32)
    hi16 = lax.bitcast_convert_type(hi, jnp.uint16).astype(jnp.uint32)
    return lax.bitcast_convert_type(lo16 | (hi16 << 16), jnp.int32)


def kernel(center_word, pos_words, neg_words, in_table, out_table):
    idx_all = jnp.concatenate(
        [pos_words, neg_words,
         jnp.zeros((B, K - N_POS - N_NEG), jnp.int32)], axis=1)
    ctr_pk = _pack_rows(jnp.take(in_table, center_word, axis=0))
    out_pk = _pack_rows(out_table)

    scores = _sc_scores(idx_all, ctr_pk, out_pk)
    scores = scores.reshape(NROW_TC, 128)

    loss = pl.pallas_call(
        _loss_body,
        grid=(NROW_TC // BLK_TC,),
        in_specs=[pl.BlockSpec((BLK_TC, 128), lambda i: (i, 0))],
        out_specs=pl.BlockSpec((1, 1), lambda i: (0, 0)),
        out_shape=jax.ShapeDtypeStruct((1, 1), jnp.float32),
    )(scores)
    return loss[0, 0]


# speedup vs baseline: 1.3781x; 1.0186x over previous
import functools

import jax
import jax.numpy as jnp
from jax import lax
from jax.experimental import pallas as pl
from jax.experimental.pallas import tpu as pltpu
from jax.experimental.pallas import tpu_sc as plsc

VOCAB = 100000
DIM = 200
DPAD = 256
WPAD = DPAD // 2
B = 4096
N_POS = 20
N_NEG = 200
K = 224
KH = K // 2
NBUF = 2
NC = 2
NS = 16
NW = NC * NS
BPW = B // NW
BBLK = 64
LANES = 16
NCHUNK = WPAD // LANES


_mesh = plsc.VectorSubcoreMesh(core_axis_name="c", subcore_axis_name="s")


@functools.partial(
    pl.kernel,
    mesh=_mesh,
    out_type=jax.ShapeDtypeStruct((B * K,), jnp.float32),
    compiler_params=pltpu.CompilerParams(
        needs_layout_passes=False, use_tc_tiling_on_sc=True),
    scratch_types=[
        pltpu.VMEM((BPW,), jnp.int32),
        pltpu.VMEM((BPW, WPAD), jnp.int32),
        pltpu.VMEM((BBLK, K), jnp.int32),
        [pltpu.VMEM((KH,), jnp.int32) for _ in range(NBUF)],
        pltpu.VMEM((NBUF, KH, WPAD), jnp.int32),
        pltpu.VMEM((BBLK * K,), jnp.float32),
        pltpu.SemaphoreType.DMA((NBUF,)),
        pltpu.SemaphoreType.DMA,
    ],
)
def _sc_scores(idx_hbm, cw_hbm, in_t_hbm, out_t_hbm, out_hbm,
               cidx_v, crows_v, kidx_v, gidx_list, rows_v, sc_v, gsem, ssem):
    gidx_v = list(gidx_list)
    wid = lax.axis_index("s") * NC + lax.axis_index("c")
    lane = lax.iota(jnp.int32, 16)
    lane0 = lane == 0

    pltpu.sync_copy(cw_hbm.at[pl.ds(wid * BPW, BPW)], cidx_v)
    pltpu.async_copy(in_t_hbm.at[cidx_v], crows_v, ssem).wait()

    def fire(b1, q):
        h1 = q & 1
        gb = gidx_v[q]
        for g in range(KH // 16):
            gb[pl.ds(g * 16, 16)] = kidx_v[b1, pl.ds(h1 * KH + g * 16, 16)]
        pltpu.async_copy(out_t_hbm.at[gb], rows_v.at[q], gsem.at[q])

    def drain(q):
        pltpu.make_async_copy(out_t_hbm.at[gidx_v[q]], rows_v.at[q],
                              gsem.at[q]).wait()

    def compute(c, b, q):
        bc = c * BBLK + b
        h = q & 1
        cvec = [plsc.bitcast(crows_v[bc, pl.ds(u * LANES, LANES)],
                             jnp.bfloat16)
                for u in range(NCHUNK)]
        obase = jnp.full((16,), b * K + h * KH, jnp.int32)

        def row(j, _):
            r0 = plsc.bitcast(rows_v[q, j, pl.ds(0, LANES)], jnp.bfloat16)
            acc = r0 * cvec[0]
            for u in range(1, NCHUNK):
                ru = plsc.bitcast(rows_v[q, j, pl.ds(u * LANES, LANES)],
                                  jnp.bfloat16)
                acc += ru * cvec[u]
            ev, od = plsc.unpack(acc, format=plsc.PackFormat.INTERLEAVED)
            s = jnp.sum(ev + od)
            plsc.store_scatter(sc_v, [obase + j],
                               jnp.full((16,), s), mask=lane0)
            return 0

        lax.fori_loop(0, KH, row, 0, unroll=4)

    for c in range(BPW // BBLK):
        base = wid * BPW + c * BBLK

        pltpu.sync_copy(idx_hbm.at[pl.ds(base, BBLK)], kidx_v)
        fire(0, 0)

        def body(b, _):
            fire(b, 1)
            drain(0)
            compute(c, b, 0)

            @pl.when(b < BBLK - 1)
            def _next():
                fire(b + 1, 0)

            drain(1)
            compute(c, b, 1)
            return 0

        lax.fori_loop(0, BBLK, body, 0)
        pltpu.sync_copy(sc_v, out_hbm.at[pl.ds(base * K, BBLK * K)])


NROW_TC = B * K // 128
BLK_TC = NROW_TC // 8


def _loss_body(scores_ref, out_ref):
    i = pl.program_id(0)

    @pl.when(i == 0)
    def _init():
        out_ref[...] = jnp.zeros((1, 1), jnp.float32)

    x = scores_ref[...]
    flat = (i * BLK_TC * 128
            + lax.broadcasted_iota(jnp.int32, x.shape, 0) * 128
            + lax.broadcasted_iota(jnp.int32, x.shape, 1))
    col = flat % K
    x = jnp.where(col < N_POS, x, -x)
    ls = jnp.where(col < N_POS + N_NEG, jax.nn.log_sigmoid(x), 0.0)
    out_ref[...] += jnp.sum(ls).reshape(1, 1)

    @pl.when(i == pl.num_programs(0) - 1)
    def _fini():
        out_ref[...] = -out_ref[...] / B


def _pack_rows(t):
    n = t.shape[0]
    lo = t[:, :WPAD].astype(jnp.bfloat16)
    hi = jnp.concatenate(
        [t[:, WPAD:].astype(jnp.bfloat16),
         jnp.zeros((n, DPAD - DIM), jnp.bfloat16)], axis=1)
    lo16 = lax.bitcast_convert_type(lo, jnp.uint16).astype(jnp.uint32)
    hi16 = lax.bitcast_convert_type(hi, jnp.uint16).astype(jnp.uint32)
    return lax.bitcast_convert_type(lo16 | (hi16 << 16), jnp.int32)


def kernel(center_word, pos_words, neg_words, in_table, out_table):
    idx_all = jnp.concatenate(
        [pos_words, neg_words,
         jnp.zeros((B, K - N_POS - N_NEG), jnp.int32)], axis=1)
    in_pk = _pack_rows(in_table)
    out_pk = _pack_rows(out_table)

    scores = _sc_scores(idx_all, center_word, in_pk, out_pk)
    scores = scores.reshape(NROW_TC, 128)

    loss = pl.pallas_call(
        _loss_body,
        grid=(NROW_TC // BLK_TC,),
        in_specs=[pl.BlockSpec((BLK_TC, 128), lambda i: (i, 0))],
        out_specs=pl.BlockSpec((1, 1), lambda i: (0, 0)),
        out_shape=jax.ShapeDtypeStruct((1, 1), jnp.float32),
    )(scores)
    return loss[0, 0]
